# Initial kernel scaffold; baseline (speedup 1.0000x reference)
import jax
import jax.numpy as jnp
from jax.experimental import pallas as pl


def kernel(x, edge_index, W, b):
    raise NotImplementedError("write your pallas kernel here")


# speedup vs baseline: 18.5789x; 18.5789x over previous
import functools

import jax
import jax.numpy as jnp
from jax import lax
from jax.experimental import pallas as pl
from jax.experimental.pallas import tpu as pltpu
from jax.experimental.pallas import tpu_sc as plsc

N = 10000
C = 128
CH = C // 2
E = 320000
NP = 10240
NC = 2
NS = 16
NT = NC * NS
K = 80
EPS = E // NS
NB = EPS // K
STR = NP // NS
RB = 1024

_mesh = plsc.VectorSubcoreMesh(core_axis_name="c", subcore_axis_name="s")

EPT = E // NT
NBD = EPT // K


@functools.partial(
    pl.kernel,
    mesh=_mesh,
    out_type=jax.ShapeDtypeStruct((NC, NP), jnp.float32),
    scratch_types=[
        pltpu.VMEM((NBD, K), jnp.int32),
        pltpu.VMEM((K,), jnp.float32),
        pltpu.VMEM((STR,), jnp.float32),
        pltpu.VMEM_SHARED((NP,), jnp.float32),
    ],
)
def _deg_kernel(tar_hbm, out_hbm, tidx_v, ones_v, zer_v, deg_sp):
    c = lax.axis_index("c")
    s = lax.axis_index("s")
    wid = c * NS + s

    def zf(i, carry):
        zer_v[pl.ds(i * 16, 16)] = jnp.zeros((16,), jnp.float32)
        return carry

    lax.fori_loop(0, STR // 16, zf, 0)

    def of(i, carry):
        ones_v[pl.ds(i * 16, 16)] = jnp.ones((16,), jnp.float32)
        return carry

    lax.fori_loop(0, K // 16, of, 0)

    pltpu.sync_copy(zer_v, deg_sp.at[pl.ds(s * STR, STR)])
    pltpu.sync_copy(tar_hbm.at[wid], tidx_v)
    plsc.subcore_barrier()

    def body(j, carry):
        pltpu.sync_copy(ones_v, deg_sp.at[tidx_v.at[j]], add=True)
        return carry

    lax.fori_loop(0, NBD, body, 0)
    plsc.subcore_barrier()
    pltpu.sync_copy(deg_sp.at[pl.ds(s * STR, STR)],
                    out_hbm.at[c, pl.ds(s * STR, STR)])


@functools.partial(
    pl.kernel,
    mesh=_mesh,
    compiler_params=pltpu.CompilerParams(use_tc_tiling_on_sc=False),
    out_type=jax.ShapeDtypeStruct((NC, NP, CH), jnp.float32),
    scratch_types=[
        pltpu.VMEM((NB, K), jnp.int32),
        pltpu.VMEM((NB, K), jnp.int32),
        pltpu.VMEM((K, CH), jnp.float32),
        pltpu.VMEM((K, CH), jnp.float32),
        pltpu.VMEM_SHARED((NP, CH), jnp.float32),
    ],
)
def _agg_kernel(src_hbm, tar_hbm, g_hbm, out_hbm,
                sidx_v, tidx_v, rows_v, zer_v, acc_sp):
    c = lax.axis_index("c")
    s = lax.axis_index("s")

    def zf(i, carry):
        for kk in range(CH // 16):
            zer_v[i, pl.ds(kk * 16, 16)] = jnp.zeros((16,), jnp.float32)
        return carry

    lax.fori_loop(0, K, zf, 0)

    def zc(i, carry):
        pltpu.sync_copy(zer_v, acc_sp.at[pl.ds(s * STR + i * K, K)])
        return carry

    lax.fori_loop(0, STR // K, zc, 0)

    pltpu.sync_copy(src_hbm.at[s], sidx_v)
    pltpu.sync_copy(tar_hbm.at[s], tidx_v)
    plsc.subcore_barrier()

    def body(j, carry):
        pltpu.sync_copy(g_hbm.at[c].at[sidx_v.at[j]], rows_v)
        pltpu.sync_copy(rows_v, acc_sp.at[tidx_v.at[j]], add=True)
        return carry

    lax.fori_loop(0, NB, body, 0)
    plsc.subcore_barrier()
    pltpu.sync_copy(acc_sp.at[pl.ds(s * STR, STR)],
                    out_hbm.at[c, pl.ds(s * STR, STR)])


def _g_body(dp_ref, x_ref, w_ref, b_ref, g_ref):
    h = jnp.dot(x_ref[...], w_ref[...],
                preferred_element_type=jnp.float32) + b_ref[...]
    deg = dp_ref[0, :] + dp_ref[1, :] + 1.0
    g = lax.rsqrt(deg)[:, None] * h
    g_ref[0, :, :] = g[:, :CH]
    g_ref[1, :, :] = g[:, CH:]


_g_call = pl.pallas_call(
    _g_body,
    grid=(NP // RB,),
    in_specs=[
        pl.BlockSpec((2, RB), lambda i: (0, i)),
        pl.BlockSpec((RB, C), lambda i: (i, 0)),
        pl.BlockSpec((C, C), lambda i: (0, 0)),
        pl.BlockSpec((1, C), lambda i: (0, 0)),
    ],
    out_specs=pl.BlockSpec((2, RB, CH), lambda i: (0, i, 0)),
    out_shape=jax.ShapeDtypeStruct((NC, NP, CH), jnp.float32),
)


def _fin_body(dp_ref, a_ref, g_ref, o_ref):
    deg = dp_ref[0, :] + dp_ref[1, :] + 1.0
    dinv = lax.rsqrt(deg)[:, None]
    o_ref[:, :CH] = dinv * (a_ref[0] + g_ref[0])
    o_ref[:, CH:] = dinv * (a_ref[1] + g_ref[1])


_fin_call = pl.pallas_call(
    _fin_body,
    grid=(NP // RB,),
    in_specs=[
        pl.BlockSpec((2, RB), lambda i: (0, i)),
        pl.BlockSpec((2, RB, CH), lambda i: (0, i, 0)),
        pl.BlockSpec((2, RB, CH), lambda i: (0, i, 0)),
    ],
    out_specs=pl.BlockSpec((RB, C), lambda i: (i, 0)),
    out_shape=jax.ShapeDtypeStruct((NP, C), jnp.float32),
)


def kernel(x, edge_index, W, b):
    tar = edge_index[0]
    src = edge_index[1]
    tar_d = tar.reshape(NT, NBD, K)
    tar_r = tar.reshape(NS, NB, K)
    src_r = src.reshape(NS, NB, K)
    x_p = jnp.concatenate(
        [x, jnp.zeros((NP - N, C), x.dtype)], axis=0)

    degp = _deg_kernel(tar_d)
    g2 = _g_call(degp, x_p, W, b.reshape(1, C))
    acc2 = _agg_kernel(src_r, tar_r, g2)
    out_p = _fin_call(degp, acc2, g2)
    return out_p[:N]


# speedup vs baseline: 32.7507x; 1.7628x over previous
import functools

import jax
import jax.numpy as jnp
from jax import lax
from jax.experimental import pallas as pl
from jax.experimental.pallas import tpu as pltpu
from jax.experimental.pallas import tpu_sc as plsc

N = 10000
C = 128
CH = C // 2
E = 320000
NP = 10240
NC = 2
NS = 16
NT = NC * NS
K = 125
EPS = E // NS
NB = EPS // K
NB2 = NB // 2
ZB = 80
STR = NP // NS
RB = 1024

_mesh = plsc.VectorSubcoreMesh(core_axis_name="c", subcore_axis_name="s")

EPT = E // NT
NBD = EPT // K


@functools.partial(
    pl.kernel,
    mesh=_mesh,
    out_type=jax.ShapeDtypeStruct((NC, NP), jnp.float32),
    scratch_types=[
        pltpu.VMEM((NBD, K), jnp.int32),
        pltpu.VMEM((128,), jnp.float32),
        pltpu.VMEM((STR,), jnp.float32),
        pltpu.VMEM_SHARED((NP,), jnp.float32),
    ],
)
def _deg_kernel(tar_hbm, out_hbm, tidx_v, ones_v, zer_v, deg_sp):
    c = lax.axis_index("c")
    s = lax.axis_index("s")
    wid = c * NS + s

    def zf(i, carry):
        zer_v[pl.ds(i * 16, 16)] = jnp.zeros((16,), jnp.float32)
        return carry

    lax.fori_loop(0, STR // 16, zf, 0)

    def of(i, carry):
        ones_v[pl.ds(i * 16, 16)] = jnp.ones((16,), jnp.float32)
        return carry

    lax.fori_loop(0, 8, of, 0)

    pltpu.sync_copy(zer_v, deg_sp.at[pl.ds(s * STR, STR)])
    pltpu.sync_copy(tar_hbm.at[wid], tidx_v)
    plsc.subcore_barrier()

    def body(j, carry):
        pltpu.sync_copy(ones_v.at[pl.ds(0, K)],
                        deg_sp.at[tidx_v.at[j]], add=True)
        return carry

    lax.fori_loop(0, NBD, body, 0)
    plsc.subcore_barrier()
    pltpu.sync_copy(deg_sp.at[pl.ds(s * STR, STR)],
                    out_hbm.at[c, pl.ds(s * STR, STR)])


@functools.partial(
    pl.kernel,
    mesh=_mesh,
    compiler_params=pltpu.CompilerParams(use_tc_tiling_on_sc=False),
    out_type=jax.ShapeDtypeStruct((NC, NP, CH), jnp.float32),
    scratch_types=[
        pltpu.VMEM((NB, K), jnp.int32),
        pltpu.VMEM((NB, K), jnp.int32),
        pltpu.VMEM((K, CH), jnp.float32),
        pltpu.VMEM((K, CH), jnp.float32),
        pltpu.VMEM((ZB, CH), jnp.float32),
        pltpu.VMEM_SHARED((NP, CH), jnp.float32),
        pltpu.SemaphoreType.DMA,
        pltpu.SemaphoreType.DMA,
    ],
)
def _agg_kernel(src_hbm, tar_hbm, g_hbm, out_hbm,
                sidx_v, tidx_v, rows0_v, rows1_v, zer_v, acc_sp,
                sem0, sem1):
    c = lax.axis_index("c")
    s = lax.axis_index("s")

    def zf(i, carry):
        for kk in range(CH // 16):
            zer_v[i, pl.ds(kk * 16, 16)] = jnp.zeros((16,), jnp.float32)
        return carry

    lax.fori_loop(0, ZB, zf, 0)

    def zc(i, carry):
        pltpu.sync_copy(zer_v, acc_sp.at[pl.ds(s * STR + i * ZB, ZB)])
        return carry

    lax.fori_loop(0, STR // ZB, zc, 0)

    pltpu.sync_copy(src_hbm.at[s], sidx_v)
    pltpu.sync_copy(tar_hbm.at[s], tidx_v)
    plsc.subcore_barrier()

    ga = g_hbm.at[c]
    pltpu.async_copy(ga.at[sidx_v.at[0]], rows0_v, sem0)
    pltpu.async_copy(ga.at[sidx_v.at[1]], rows1_v, sem1)

    def body(m, carry):
        j = 2 * m
        pltpu.make_async_copy(ga.at[sidx_v.at[j]], rows0_v, sem0).wait()
        pltpu.sync_copy(rows0_v, acc_sp.at[tidx_v.at[j]], add=True)
        pltpu.async_copy(ga.at[sidx_v.at[j + 2]], rows0_v, sem0)
        pltpu.make_async_copy(ga.at[sidx_v.at[j + 1]], rows1_v, sem1).wait()
        pltpu.sync_copy(rows1_v, acc_sp.at[tidx_v.at[j + 1]], add=True)
        pltpu.async_copy(ga.at[sidx_v.at[j + 3]], rows1_v, sem1)
        return carry

    lax.fori_loop(0, NB2 - 1, body, 0)

    j = NB - 2
    pltpu.make_async_copy(ga.at[sidx_v.at[j]], rows0_v, sem0).wait()
    pltpu.sync_copy(rows0_v, acc_sp.at[tidx_v.at[j]], add=True)
    pltpu.make_async_copy(ga.at[sidx_v.at[j + 1]], rows1_v, sem1).wait()
    pltpu.sync_copy(rows1_v, acc_sp.at[tidx_v.at[j + 1]], add=True)
    plsc.subcore_barrier()
    pltpu.sync_copy(acc_sp.at[pl.ds(s * STR, STR)],
                    out_hbm.at[c, pl.ds(s * STR, STR)])


def _g_body(dp_ref, x_ref, w_ref, b_ref, g_ref):
    h = jnp.dot(x_ref[...], w_ref[...],
                preferred_element_type=jnp.float32) + b_ref[...]
    deg = dp_ref[0, :] + dp_ref[1, :] + 1.0
    g = lax.rsqrt(deg)[:, None] * h
    g_ref[0, :, :] = g[:, :CH]
    g_ref[1, :, :] = g[:, CH:]


_g_call = pl.pallas_call(
    _g_body,
    grid=(NP // RB,),
    in_specs=[
        pl.BlockSpec((2, RB), lambda i: (0, i)),
        pl.BlockSpec((RB, C), lambda i: (i, 0)),
        pl.BlockSpec((C, C), lambda i: (0, 0)),
        pl.BlockSpec((1, C), lambda i: (0, 0)),
    ],
    out_specs=pl.BlockSpec((2, RB, CH), lambda i: (0, i, 0)),
    out_shape=jax.ShapeDtypeStruct((NC, NP, CH), jnp.float32),
)


def _fin_body(dp_ref, a_ref, g_ref, o_ref):
    deg = dp_ref[0, :] + dp_ref[1, :] + 1.0
    dinv = lax.rsqrt(deg)[:, None]
    o_ref[:, :CH] = dinv * (a_ref[0] + g_ref[0])
    o_ref[:, CH:] = dinv * (a_ref[1] + g_ref[1])


_fin_call = pl.pallas_call(
    _fin_body,
    grid=(NP // RB,),
    in_specs=[
        pl.BlockSpec((2, RB), lambda i: (0, i)),
        pl.BlockSpec((2, RB, CH), lambda i: (0, i, 0)),
        pl.BlockSpec((2, RB, CH), lambda i: (0, i, 0)),
    ],
    out_specs=pl.BlockSpec((RB, C), lambda i: (i, 0)),
    out_shape=jax.ShapeDtypeStruct((NP, C), jnp.float32),
)


def kernel(x, edge_index, W, b):
    tar = edge_index[0]
    src = edge_index[1]
    tar_d = tar.reshape(NT, NBD, K)
    tar_r = tar.reshape(NS, NB, K)
    src_r = src.reshape(NS, NB, K)
    x_p = jnp.concatenate(
        [x, jnp.zeros((NP - N, C), x.dtype)], axis=0)

    degp = _deg_kernel(tar_d)
    g2 = _g_call(degp, x_p, W, b.reshape(1, C))
    acc2 = _agg_kernel(src_r, tar_r, g2)
    out_p = _fin_call(degp, acc2, g2)
    return out_p[:N]


# speedup vs baseline: 38.6542x; 1.1803x over previous
import functools

import jax
import jax.numpy as jnp
from jax import lax
from jax.experimental import pallas as pl
from jax.experimental.pallas import tpu as pltpu
from jax.experimental.pallas import tpu_sc as plsc

N = 10000
C = 128
CH = C // 2
E = 320000
NP = 10240
NC = 2
NS = 16
NT = NC * NS
K = 125
EPS = E // NS
NB = EPS // K
NB2 = NB // 2
ZB = 80
STR = NP // NS
RB = 1024

_mesh = plsc.VectorSubcoreMesh(core_axis_name="c", subcore_axis_name="s")

NBD = NB // NC


@functools.partial(
    pl.kernel,
    mesh=_mesh,
    out_type=jax.ShapeDtypeStruct((NC, NP), jnp.float32),
    scratch_types=[
        pltpu.VMEM((NBD, K), jnp.int32),
        pltpu.VMEM((128,), jnp.float32),
        pltpu.VMEM((STR,), jnp.float32),
        pltpu.VMEM_SHARED((NP,), jnp.float32),
    ],
)
def _deg_kernel(tar_hbm, out_hbm, tidx_v, ones_v, zer_v, deg_sp):
    c = lax.axis_index("c")
    s = lax.axis_index("s")

    def zf(i, carry):
        zer_v[pl.ds(i * 16, 16)] = jnp.zeros((16,), jnp.float32)
        return carry

    lax.fori_loop(0, STR // 16, zf, 0)

    def of(i, carry):
        ones_v[pl.ds(i * 16, 16)] = jnp.ones((16,), jnp.float32)
        return carry

    lax.fori_loop(0, 8, of, 0)

    pltpu.sync_copy(zer_v, deg_sp.at[pl.ds(s * STR, STR)])
    pltpu.sync_copy(tar_hbm.at[s, pl.ds(c * NBD, NBD)], tidx_v)
    plsc.subcore_barrier()

    def body(j, carry):
        pltpu.sync_copy(ones_v.at[pl.ds(0, K)],
                        deg_sp.at[tidx_v.at[j]], add=True)
        return carry

    lax.fori_loop(0, NBD, body, 0)
    plsc.subcore_barrier()
    pltpu.sync_copy(deg_sp.at[pl.ds(s * STR, STR)],
                    out_hbm.at[c, pl.ds(s * STR, STR)])


@functools.partial(
    pl.kernel,
    mesh=_mesh,
    compiler_params=pltpu.CompilerParams(use_tc_tiling_on_sc=False),
    out_type=jax.ShapeDtypeStruct((NC, NP, CH), jnp.float32),
    scratch_types=[
        pltpu.VMEM((NB, K), jnp.int32),
        pltpu.VMEM((NB, K), jnp.int32),
        pltpu.VMEM((K, CH), jnp.float32),
        pltpu.VMEM((K, CH), jnp.float32),
        pltpu.VMEM((K, CH), jnp.float32),
        pltpu.VMEM((K, CH), jnp.float32),
        pltpu.VMEM((ZB, CH), jnp.float32),
        pltpu.VMEM_SHARED((NP, CH), jnp.float32),
        pltpu.SemaphoreType.DMA,
        pltpu.SemaphoreType.DMA,
        pltpu.SemaphoreType.DMA,
        pltpu.SemaphoreType.DMA,
    ],
)
def _agg_kernel(src_hbm, tar_hbm, g_hbm, out_hbm,
                sidx_v, tidx_v, rows0_v, rows1_v, rows2_v, rows3_v,
                zer_v, acc_sp, sem0, sem1, sem2, sem3):
    c = lax.axis_index("c")
    s = lax.axis_index("s")

    def zf(i, carry):
        for kk in range(CH // 16):
            zer_v[i, pl.ds(kk * 16, 16)] = jnp.zeros((16,), jnp.float32)
        return carry

    lax.fori_loop(0, ZB, zf, 0)

    def zc(i, carry):
        pltpu.sync_copy(zer_v, acc_sp.at[pl.ds(s * STR + i * ZB, ZB)])
        return carry

    lax.fori_loop(0, STR // ZB, zc, 0)

    pltpu.sync_copy(src_hbm.at[s], sidx_v)
    pltpu.sync_copy(tar_hbm.at[s], tidx_v)
    plsc.subcore_barrier()

    ga = g_hbm.at[c]
    bufs = [(rows0_v, sem0), (rows1_v, sem1), (rows2_v, sem2),
            (rows3_v, sem3)]
    for b, (rv, sm) in enumerate(bufs):
        pltpu.async_copy(ga.at[sidx_v.at[b]], rv, sm)

    def body(m, carry):
        j = 4 * m
        for b, (rv, sm) in enumerate(bufs):
            pltpu.make_async_copy(ga.at[sidx_v.at[j + b]], rv, sm).wait()
            pltpu.sync_copy(rv, acc_sp.at[tidx_v.at[j + b]], add=True)
            pltpu.async_copy(ga.at[sidx_v.at[j + b + 4]], rv, sm)
        return carry

    lax.fori_loop(0, NB // 4 - 1, body, 0)

    j = NB - 4
    for b, (rv, sm) in enumerate(bufs):
        pltpu.make_async_copy(ga.at[sidx_v.at[j + b]], rv, sm).wait()
        pltpu.sync_copy(rv, acc_sp.at[tidx_v.at[j + b]], add=True)
    plsc.subcore_barrier()
    pltpu.sync_copy(acc_sp.at[pl.ds(s * STR, STR)],
                    out_hbm.at[c, pl.ds(s * STR, STR)])


def _g_body(dp_ref, x_ref, w_ref, b_ref, g_ref):
    h = jnp.dot(x_ref[...], w_ref[...],
                preferred_element_type=jnp.float32) + b_ref[...]
    deg = dp_ref[0, :] + dp_ref[1, :] + 1.0
    g = lax.rsqrt(deg)[:, None] * h
    g_ref[0, :, :] = g[:, :CH]
    g_ref[1, :, :] = g[:, CH:]


_g_call = pl.pallas_call(
    _g_body,
    grid=(NP // RB,),
    in_specs=[
        pl.BlockSpec((2, RB), lambda i: (0, i)),
        pl.BlockSpec((RB, C), lambda i: (i, 0)),
        pl.BlockSpec((C, C), lambda i: (0, 0)),
        pl.BlockSpec((1, C), lambda i: (0, 0)),
    ],
    out_specs=pl.BlockSpec((2, RB, CH), lambda i: (0, i, 0)),
    out_shape=jax.ShapeDtypeStruct((NC, NP, CH), jnp.float32),
)


def _fin_body(dp_ref, a_ref, g_ref, o_ref):
    deg = dp_ref[0, :] + dp_ref[1, :] + 1.0
    dinv = lax.rsqrt(deg)[:, None]
    o_ref[:, :CH] = dinv * (a_ref[0] + g_ref[0])
    o_ref[:, CH:] = dinv * (a_ref[1] + g_ref[1])


_fin_call = pl.pallas_call(
    _fin_body,
    grid=(NP // RB,),
    in_specs=[
        pl.BlockSpec((2, RB), lambda i: (0, i)),
        pl.BlockSpec((2, RB, CH), lambda i: (0, i, 0)),
        pl.BlockSpec((2, RB, CH), lambda i: (0, i, 0)),
    ],
    out_specs=pl.BlockSpec((RB, C), lambda i: (i, 0)),
    out_shape=jax.ShapeDtypeStruct((NP, C), jnp.float32),
)


def kernel(x, edge_index, W, b):
    tar = edge_index[0]
    src = edge_index[1]
    tar_r = tar.reshape(NS, NB, K)
    src_r = src.reshape(NS, NB, K)
    x_p = jnp.concatenate(
        [x, jnp.zeros((NP - N, C), x.dtype)], axis=0)

    degp = _deg_kernel(tar_r)
    g2 = _g_call(degp, x_p, W, b.reshape(1, C))
    acc2 = _agg_kernel(src_r, tar_r, g2)
    out_p = _fin_call(degp, acc2, g2)
    return out_p[:N]
